# Initial kernel scaffold; baseline (speedup 1.0000x reference)
import jax
import jax.numpy as jnp
from jax.experimental import pallas as pl


def kernel(logits, Match_htr, match_rels, match_rels_mask):
    raise NotImplementedError("write your pallas kernel here")


# speedup vs baseline: 1.9383x; 1.9383x over previous
import functools

import jax
import jax.numpy as jnp
from jax import lax
from jax.experimental import pallas as pl
from jax.experimental.pallas import tpu as pltpu
from jax.experimental.pallas import tpu_sc as plsc

N = 2048
C = 1024
K = 20
NUM_WORKERS = 32
ROWS_PER_WORKER = N // NUM_WORKERS
CHUNK = 16
TC_BLK = 256


def _sc_top1(logits_flat_hbm, rels_flat_hbm, mask_flat_hbm, t_hbm, m_hbm,
             log_v, rels_v, mask_v, tv, mv):
    wid = lax.axis_index("s") * 2 + lax.axis_index("c")
    for ci in range(ROWS_PER_WORKER // CHUNK):
        chunk_id = wid * (ROWS_PER_WORKER // CHUNK) + ci
        base = chunk_id * CHUNK
        pltpu.sync_copy(logits_flat_hbm.at[pl.ds(base * C, CHUNK * C)], log_v)
        pltpu.sync_copy(rels_flat_hbm.at[pl.ds(chunk_id * K * CHUNK, K * CHUNK)],
                        rels_v)
        pltpu.sync_copy(mask_flat_hbm.at[pl.ds(chunk_id * K * CHUNK, K * CHUNK)],
                        mask_v)
        row_off = lax.iota(jnp.int32, CHUNK) * C
        best_s = jnp.full((CHUNK,), -1.0, jnp.float32)
        best_g = jnp.zeros((CHUNK,), jnp.float32)
        best_m = jnp.zeros((CHUNK,), jnp.float32)
        for k in range(K):
            idx = rels_v[pl.ds(k * CHUNK, CHUNK)]
            g = plsc.load_gather(log_v, [row_off + idx])
            mk = mask_v[pl.ds(k * CHUNK, CHUNK)]
            s = mk / (1.0 + jnp.exp(-g))
            upd = s > best_s
            best_s = jnp.where(upd, s, best_s)
            best_g = jnp.where(upd, g, best_g)
            best_m = jnp.where(upd, mk, best_m)
        tv[...] = best_g
        mv[...] = best_m
        pltpu.sync_copy(tv, t_hbm.at[pl.ds(base, CHUNK)])
        pltpu.sync_copy(mv, m_hbm.at[pl.ds(base, CHUNK)])


_sc_call = functools.partial(
    pl.kernel,
    out_type=(jax.ShapeDtypeStruct((N,), jnp.float32),
              jax.ShapeDtypeStruct((N,), jnp.float32)),
    mesh=plsc.VectorSubcoreMesh(core_axis_name="c", subcore_axis_name="s"),
    compiler_params=pltpu.CompilerParams(needs_layout_passes=False),
    scratch_types=[
        pltpu.VMEM((CHUNK * C,), jnp.float32),
        pltpu.VMEM((K * CHUNK,), jnp.int32),
        pltpu.VMEM((K * CHUNK,), jnp.float32),
        pltpu.VMEM((CHUNK,), jnp.float32),
        pltpu.VMEM((CHUNK,), jnp.float32),
    ],
)(_sc_top1)


def _tc_loss_body(logits_ref, htr_ref, t_ref, m_ref, out_ref, acc_ref):
    i = pl.program_id(0)

    @pl.when(i == 0)
    def _init():
        acc_ref[0] = 0.0
        acc_ref[1] = 0.0

    lg = logits_ref[...]
    cmp = (lg >= t_ref[...]).astype(jnp.float32)
    rank = jnp.sum(cmp, axis=1, keepdims=True)
    neg = jnp.sum(cmp * (1.0 - htr_ref[...]), axis=1, keepdims=True)
    neg = jnp.maximum(neg, 0.1)
    mcol = m_ref[...]
    acc_ref[0] += jnp.sum(mcol * neg / rank)
    acc_ref[1] += jnp.sum(mcol)

    @pl.when(i == (N // TC_BLK) - 1)
    def _fin():
        out_ref[0, 0] = acc_ref[0] / acc_ref[1]


_tc_call = pl.pallas_call(
    _tc_loss_body,
    grid=(N // TC_BLK,),
    in_specs=[
        pl.BlockSpec((TC_BLK, C), lambda i: (i, 0)),
        pl.BlockSpec((TC_BLK, C), lambda i: (i, 0)),
        pl.BlockSpec((TC_BLK, 1), lambda i: (i, 0)),
        pl.BlockSpec((TC_BLK, 1), lambda i: (i, 0)),
    ],
    out_specs=pl.BlockSpec(memory_space=pltpu.SMEM),
    out_shape=jax.ShapeDtypeStruct((1, 1), jnp.float32),
    scratch_shapes=[pltpu.SMEM((2,), jnp.float32)],
)


def kernel(logits, Match_htr, match_rels, match_rels_mask):
    rels_flat = (match_rels.astype(jnp.int32)
                 .reshape(N // CHUNK, CHUNK, K).transpose(0, 2, 1).reshape(-1))
    mask_flat = (match_rels_mask.astype(jnp.float32)
                 .reshape(N // CHUNK, CHUNK, K).transpose(0, 2, 1).reshape(-1))
    t, m = _sc_call(logits.reshape(-1), rels_flat, mask_flat)
    loss = _tc_call(logits, Match_htr,
                    t.reshape(N, 1), m.reshape(N, 1))
    return loss[0, 0]


# speedup vs baseline: 2.5246x; 1.3024x over previous
import functools

import jax
import jax.numpy as jnp
from jax import lax
from jax.experimental import pallas as pl
from jax.experimental.pallas import tpu as pltpu
from jax.experimental.pallas import tpu_sc as plsc

N = 2048
C = 1024
K = 20
NUM_WORKERS = 32
ROWS_PER_WORKER = N // NUM_WORKERS
CHUNK = 16
NCHUNK = ROWS_PER_WORKER // CHUNK
TC_BLK = 256


def _sc_top1(logits_hbm, rels_hbm, mask_hbm, t_hbm, m_hbm,
             log_v0, log_v1, rels_v, mask_v, tv, mv, sem0, sem1):
    wid = lax.axis_index("s") * 2 + lax.axis_index("c")
    tbase = wid * ROWS_PER_WORKER
    pltpu.sync_copy(rels_hbm.at[pl.ds(tbase, ROWS_PER_WORKER), :], rels_v)
    pltpu.sync_copy(mask_hbm.at[pl.ds(tbase, ROWS_PER_WORKER), :], mask_v)
    logs = [log_v0, log_v1]
    sems = [sem0, sem1]
    cps = [None, None]
    cps[0] = pltpu.async_copy(
        logits_hbm.at[pl.ds(tbase, CHUNK), :], logs[0], sems[0])
    for ci in range(NCHUNK):
        if ci + 1 < NCHUNK:
            nb = (ci + 1) % 2
            cps[nb] = pltpu.async_copy(
                logits_hbm.at[pl.ds(tbase + (ci + 1) * CHUNK, CHUNK), :],
                logs[nb], sems[nb])
        cps[ci % 2].wait()
        log_v = logs[ci % 2]
        rows = lax.iota(jnp.int32, CHUNK)
        best_s = jnp.full((CHUNK,), -1.0, jnp.float32)
        best_g = jnp.zeros((CHUNK,), jnp.float32)
        best_m = jnp.zeros((CHUNK,), jnp.float32)
        for k in range(K):
            kk = jnp.full((CHUNK,), k, jnp.int32)
            rk = plsc.load_gather(rels_v, [ci * CHUNK + rows, kk])
            mk = plsc.load_gather(mask_v, [ci * CHUNK + rows, kk])
            g = plsc.load_gather(log_v, [rows, rk])
            s = mk / (1.0 + jnp.exp(-g))
            upd = s > best_s
            best_s = jnp.where(upd, s, best_s)
            best_g = jnp.where(upd, g, best_g)
            best_m = jnp.where(upd, mk, best_m)
        tv[pl.ds(ci * CHUNK, CHUNK)] = best_g
        mv[pl.ds(ci * CHUNK, CHUNK)] = best_m
    pltpu.sync_copy(tv, t_hbm.at[pl.ds(tbase, ROWS_PER_WORKER)])
    pltpu.sync_copy(mv, m_hbm.at[pl.ds(tbase, ROWS_PER_WORKER)])


_sc_call = functools.partial(
    pl.kernel,
    out_type=(jax.ShapeDtypeStruct((N,), jnp.float32),
              jax.ShapeDtypeStruct((N,), jnp.float32)),
    mesh=plsc.VectorSubcoreMesh(core_axis_name="c", subcore_axis_name="s"),
    compiler_params=pltpu.CompilerParams(needs_layout_passes=False),
    scratch_types=[
        pltpu.VMEM((CHUNK, C), jnp.float32),
        pltpu.VMEM((CHUNK, C), jnp.float32),
        pltpu.VMEM((ROWS_PER_WORKER, K), jnp.int32),
        pltpu.VMEM((ROWS_PER_WORKER, K), jnp.float32),
        pltpu.VMEM((ROWS_PER_WORKER,), jnp.float32),
        pltpu.VMEM((ROWS_PER_WORKER,), jnp.float32),
        pltpu.SemaphoreType.DMA,
        pltpu.SemaphoreType.DMA,
    ],
)(_sc_top1)


def _tc_loss_body(logits_ref, htr_ref, t_ref, m_ref, out_ref, acc_ref):
    i = pl.program_id(0)

    @pl.when(i == 0)
    def _init():
        acc_ref[0] = 0.0
        acc_ref[1] = 0.0

    lg = logits_ref[...]
    cmp = (lg >= t_ref[...]).astype(jnp.float32)
    rank = jnp.sum(cmp, axis=1, keepdims=True)
    neg = jnp.sum(cmp * (1.0 - htr_ref[...]), axis=1, keepdims=True)
    neg = jnp.maximum(neg, 0.1)
    mcol = m_ref[...]
    acc_ref[0] += jnp.sum(mcol * neg / rank)
    acc_ref[1] += jnp.sum(mcol)

    @pl.when(i == (N // TC_BLK) - 1)
    def _fin():
        out_ref[0, 0] = acc_ref[0] / acc_ref[1]


_tc_call = pl.pallas_call(
    _tc_loss_body,
    grid=(N // TC_BLK,),
    in_specs=[
        pl.BlockSpec((TC_BLK, C), lambda i: (i, 0)),
        pl.BlockSpec((TC_BLK, C), lambda i: (i, 0)),
        pl.BlockSpec((TC_BLK, 1), lambda i: (i, 0)),
        pl.BlockSpec((TC_BLK, 1), lambda i: (i, 0)),
    ],
    out_specs=pl.BlockSpec(memory_space=pltpu.SMEM),
    out_shape=jax.ShapeDtypeStruct((1, 1), jnp.float32),
    scratch_shapes=[pltpu.SMEM((2,), jnp.float32)],
)


def kernel(logits, Match_htr, match_rels, match_rels_mask):
    t, m = _sc_call(logits, match_rels.astype(jnp.int32),
                    match_rels_mask.astype(jnp.float32))
    loss = _tc_call(logits, Match_htr,
                    t.reshape(N, 1), m.reshape(N, 1))
    return loss[0, 0]


# speedup vs baseline: 2.9150x; 1.1546x over previous
import functools

import jax
import jax.numpy as jnp
from jax import lax
from jax.experimental import pallas as pl
from jax.experimental.pallas import tpu as pltpu
from jax.experimental.pallas import tpu_sc as plsc

N = 2048
C = 1024
K = 20
NUM_WORKERS = 32
ROWS_PER_WORKER = N // NUM_WORKERS
CHUNK = 16
NCHUNK = ROWS_PER_WORKER // CHUNK
COLBLK = 128
TC_BLK = 256


def _sc_top1(logits_hbm, relsT_hbm, maskT_hbm, t_hbm, m_hbm,
             log_v0, log_v1, rels_v, mask_v, tv, mv, sem0, sem1):
    wid = lax.axis_index("s") * 2 + lax.axis_index("c")
    tbase = wid * ROWS_PER_WORKER
    c128 = (tbase // COLBLK) * COLBLK
    coff = tbase - c128
    pltpu.sync_copy(relsT_hbm.at[:, pl.ds(c128, COLBLK)], rels_v)
    pltpu.sync_copy(maskT_hbm.at[:, pl.ds(c128, COLBLK)], mask_v)
    logs = [log_v0, log_v1]
    sems = [sem0, sem1]
    cps = [None, None]
    cps[0] = pltpu.async_copy(
        logits_hbm.at[pl.ds(tbase, CHUNK), :], logs[0], sems[0])
    for ci in range(NCHUNK):
        if ci + 1 < NCHUNK:
            nb = (ci + 1) % 2
            cps[nb] = pltpu.async_copy(
                logits_hbm.at[pl.ds(tbase + (ci + 1) * CHUNK, CHUNK), :],
                logs[nb], sems[nb])
        cps[ci % 2].wait()
        log_v = logs[ci % 2]
        rows = lax.iota(jnp.int32, CHUNK)
        best_s = jnp.full((CHUNK,), -1.0, jnp.float32)
        best_g = jnp.zeros((CHUNK,), jnp.float32)
        best_m = jnp.zeros((CHUNK,), jnp.float32)
        for k in range(K):
            rk = rels_v[k, pl.ds(coff + ci * CHUNK, CHUNK)]
            mk = mask_v[k, pl.ds(coff + ci * CHUNK, CHUNK)]
            g = plsc.load_gather(log_v, [rows, rk])
            s = mk / (1.0 + jnp.exp(-g))
            upd = s > best_s
            best_s = jnp.where(upd, s, best_s)
            best_g = jnp.where(upd, g, best_g)
            best_m = jnp.where(upd, mk, best_m)
        zz = jnp.zeros((CHUNK,), jnp.int32)
        plsc.store_scatter(tv, [ci * CHUNK + rows, zz], best_g)
        plsc.store_scatter(mv, [ci * CHUNK + rows, zz], best_m)
    pltpu.sync_copy(tv, t_hbm.at[pl.ds(tbase, ROWS_PER_WORKER), :])
    pltpu.sync_copy(mv, m_hbm.at[pl.ds(tbase, ROWS_PER_WORKER), :])


_sc_call = functools.partial(
    pl.kernel,
    out_type=(jax.ShapeDtypeStruct((N, 1), jnp.float32),
              jax.ShapeDtypeStruct((N, 1), jnp.float32)),
    mesh=plsc.VectorSubcoreMesh(core_axis_name="c", subcore_axis_name="s"),
    compiler_params=pltpu.CompilerParams(needs_layout_passes=False),
    scratch_types=[
        pltpu.VMEM((CHUNK, C), jnp.float32),
        pltpu.VMEM((CHUNK, C), jnp.float32),
        pltpu.VMEM((K, COLBLK), jnp.int32),
        pltpu.VMEM((K, COLBLK), jnp.float32),
        pltpu.VMEM((ROWS_PER_WORKER, 1), jnp.float32),
        pltpu.VMEM((ROWS_PER_WORKER, 1), jnp.float32),
        pltpu.SemaphoreType.DMA,
        pltpu.SemaphoreType.DMA,
    ],
)(_sc_top1)


def _tc_loss_body(logits_ref, htr_ref, t_ref, m_ref, out_ref, acc_ref):
    i = pl.program_id(0)

    @pl.when(i == 0)
    def _init():
        acc_ref[0] = 0.0
        acc_ref[1] = 0.0

    lg = logits_ref[...]
    cmp = (lg >= t_ref[...]).astype(jnp.float32)
    rank = jnp.sum(cmp, axis=1, keepdims=True)
    neg = jnp.sum(cmp * (1.0 - htr_ref[...]), axis=1, keepdims=True)
    neg = jnp.maximum(neg, 0.1)
    mcol = m_ref[...]
    acc_ref[0] += jnp.sum(mcol * neg / rank)
    acc_ref[1] += jnp.sum(mcol)

    @pl.when(i == (N // TC_BLK) - 1)
    def _fin():
        out_ref[0, 0] = acc_ref[0] / acc_ref[1]


_tc_call = pl.pallas_call(
    _tc_loss_body,
    grid=(N // TC_BLK,),
    in_specs=[
        pl.BlockSpec((TC_BLK, C), lambda i: (i, 0)),
        pl.BlockSpec((TC_BLK, C), lambda i: (i, 0)),
        pl.BlockSpec((TC_BLK, 1), lambda i: (i, 0)),
        pl.BlockSpec((TC_BLK, 1), lambda i: (i, 0)),
    ],
    out_specs=pl.BlockSpec(memory_space=pltpu.SMEM),
    out_shape=jax.ShapeDtypeStruct((1, 1), jnp.float32),
    scratch_shapes=[pltpu.SMEM((2,), jnp.float32)],
)


def kernel(logits, Match_htr, match_rels, match_rels_mask):
    t, m = _sc_call(logits, match_rels.astype(jnp.int32).T,
                    match_rels_mask.astype(jnp.float32).T)
    loss = _tc_call(logits, Match_htr, t, m)
    return loss[0, 0]
